# baseline (speedup 1.0000x reference)
import jax
import jax.numpy as jnp
from jax.experimental import pallas as pl
from jax.experimental.pallas import tpu as pltpu

_ROWS_PER_BLOCK = 2048


def _fma_body(rows_per_b, idx_ref, site_ref, mask_ref, x_ref, o_ref):
    rb = x_ref.shape[0]
    if rb <= rows_per_b:
        b = (pl.program_id(0) * rb) // rows_per_b
        j = idx_ref[b]
        o_ref[...] = x_ref[...] * mask_ref[j] + site_ref[j]
    else:
        nb = rb // rows_per_b
        b0 = pl.program_id(0) * nb
        for k in range(nb):
            j = idx_ref[b0 + k]
            sl = pl.ds(k * rows_per_b, rows_per_b)
            o_ref[sl, :] = x_ref[sl, :] * mask_ref[j] + site_ref[j]


def kernel(inputs, available_injection_sites, masks):
    B, H, W, C = inputs.shape
    n = available_injection_sites.shape[0]
    idx = jax.random.randint(jax.random.key(22), (B,), 0, n).astype(jnp.int32)
    sites = available_injection_sites.reshape(n)
    msk = masks.reshape(n)

    rows_per_b = H * W
    rb = _ROWS_PER_BLOCK
    if not (rb % rows_per_b == 0 or rows_per_b % rb == 0):
        rb = rows_per_b
    total = B * rows_per_b
    x = inputs.reshape(total, C)

    import functools
    out = pl.pallas_call(
        functools.partial(_fma_body, rows_per_b),
        grid=(total // rb,),
        in_specs=[
            pl.BlockSpec(memory_space=pltpu.SMEM),
            pl.BlockSpec(memory_space=pltpu.SMEM),
            pl.BlockSpec(memory_space=pltpu.SMEM),
            pl.BlockSpec((rb, C), lambda i: (i, 0)),
        ],
        out_specs=pl.BlockSpec((rb, C), lambda i: (i, 0)),
        out_shape=jax.ShapeDtypeStruct((total, C), inputs.dtype),
        compiler_params=pltpu.CompilerParams(
            dimension_semantics=("parallel",),
        ),
    )(idx, sites, msk, x)
    return out.reshape(B, H, W, C)


# speedup vs baseline: 1.0186x; 1.0186x over previous
import jax
import jax.numpy as jnp
from jax.experimental import pallas as pl
from jax.experimental.pallas import tpu as pltpu

_ROWS_PER_BLOCK = 4096


def _fma_body(rows_per_b, idx_ref, site_ref, mask_ref, x_ref, o_ref):
    rb = x_ref.shape[0]
    if rb <= rows_per_b:
        b = (pl.program_id(0) * rb) // rows_per_b
        j = idx_ref[b]
        o_ref[...] = x_ref[...] * mask_ref[j] + site_ref[j]
    else:
        nb = rb // rows_per_b
        b0 = pl.program_id(0) * nb
        for k in range(nb):
            j = idx_ref[b0 + k]
            sl = pl.ds(k * rows_per_b, rows_per_b)
            o_ref[sl, :] = x_ref[sl, :] * mask_ref[j] + site_ref[j]


def kernel(inputs, available_injection_sites, masks):
    B, H, W, C = inputs.shape
    n = available_injection_sites.shape[0]
    idx = jax.random.randint(jax.random.key(22), (B,), 0, n).astype(jnp.int32)
    sites = available_injection_sites.reshape(n)
    msk = masks.reshape(n)

    rows_per_b = H * W
    rb = _ROWS_PER_BLOCK
    if not (rb % rows_per_b == 0 or rows_per_b % rb == 0):
        rb = rows_per_b
    total = B * rows_per_b
    x = inputs.reshape(total, C)

    import functools
    out = pl.pallas_call(
        functools.partial(_fma_body, rows_per_b),
        grid=(total // rb,),
        in_specs=[
            pl.BlockSpec(memory_space=pltpu.SMEM),
            pl.BlockSpec(memory_space=pltpu.SMEM),
            pl.BlockSpec(memory_space=pltpu.SMEM),
            pl.BlockSpec((rb, C), lambda i: (i, 0)),
        ],
        out_specs=pl.BlockSpec((rb, C), lambda i: (i, 0)),
        out_shape=jax.ShapeDtypeStruct((total, C), inputs.dtype),
        compiler_params=pltpu.CompilerParams(
            dimension_semantics=("parallel",),
        ),
    )(idx, sites, msk, x)
    return out.reshape(B, H, W, C)


# speedup vs baseline: 1.0873x; 1.0675x over previous
import jax
import jax.numpy as jnp
from jax.experimental import pallas as pl
from jax.experimental.pallas import tpu as pltpu

_ROWS_PER_BLOCK = 4096


def _fma_body(rows_per_b, idx_ref, site_ref, mask_ref, x_ref, o_ref):
    rb = x_ref.shape[0]
    if rb <= rows_per_b:
        b = (pl.program_id(0) * rb) // rows_per_b
        j = idx_ref[b]
        o_ref[...] = x_ref[...] * mask_ref[j] + site_ref[j]
    else:
        nb = rb // rows_per_b
        b0 = pl.program_id(0) * nb
        for k in range(nb):
            j = idx_ref[b0 + k]
            sl = pl.ds(k * rows_per_b, rows_per_b)
            o_ref[sl, :] = x_ref[sl, :] * mask_ref[j] + site_ref[j]


def kernel(inputs, available_injection_sites, masks):
    B, H, W, C = inputs.shape
    n = available_injection_sites.shape[0]
    with jax.ensure_compile_time_eval():
        idx = jax.random.randint(
            jax.random.key(22), (B,), 0, n).astype(jnp.int32)
    sites = available_injection_sites.reshape(n)
    msk = masks.reshape(n)

    rows_per_b = H * W
    rb = _ROWS_PER_BLOCK
    if not (rb % rows_per_b == 0 or rows_per_b % rb == 0):
        rb = rows_per_b
    total = B * rows_per_b
    x = inputs.reshape(total, C)

    import functools
    out = pl.pallas_call(
        functools.partial(_fma_body, rows_per_b),
        grid=(total // rb,),
        in_specs=[
            pl.BlockSpec(memory_space=pltpu.SMEM),
            pl.BlockSpec(memory_space=pltpu.SMEM),
            pl.BlockSpec(memory_space=pltpu.SMEM),
            pl.BlockSpec((rb, C), lambda i: (i, 0)),
        ],
        out_specs=pl.BlockSpec((rb, C), lambda i: (i, 0)),
        out_shape=jax.ShapeDtypeStruct((total, C), inputs.dtype),
        compiler_params=pltpu.CompilerParams(
            dimension_semantics=("parallel",),
        ),
    )(idx, sites, msk, x)
    return out.reshape(B, H, W, C)
